# baseline (speedup 1.0000x reference)
import functools

import jax
import jax.numpy as jnp
from jax import lax
from jax.experimental import pallas as pl
from jax.experimental.pallas import tpu as pltpu
from jax.experimental.pallas import tpu_sc as plsc

_NB = 3
_CR = 640


def _make_sc_gather(V, D, N, NW, NB, CR):
    mesh = plsc.VectorSubcoreMesh(core_axis_name="c", subcore_axis_name="s")
    info = plsc.get_sparse_core_info()
    NC = info.num_cores
    n_per_w = N // NW
    n_chunks = n_per_w // CR

    @functools.partial(
        pl.kernel,
        mesh=mesh,
        compiler_params=pltpu.CompilerParams(use_tc_tiling_on_sc=False),
        out_type=jax.ShapeDtypeStruct((N, D), jnp.float32),
        scratch_types=[
            pltpu.VMEM((n_per_w,), jnp.int32),
            pltpu.VMEM((NB, CR, D), jnp.float32),
            pltpu.SemaphoreType.DMA((NB,)),
            pltpu.SemaphoreType.DMA((NB,)),
        ],
    )
    def gather(idx_hbm, table_hbm, out_hbm, idx_v, rows_v, gsem, ssem):
        wid = lax.axis_index("s") * NC + lax.axis_index("c")
        base = wid * n_per_w
        pltpu.sync_copy(idx_hbm.at[pl.ds(base, n_per_w)], idx_v)

        def g_start(b, j):
            pltpu.async_copy(
                table_hbm.at[idx_v.at[pl.ds(j * CR, CR)]],
                rows_v.at[b],
                gsem.at[b],
            )

        def g_wait(b):
            pltpu.make_async_copy(
                table_hbm.at[idx_v.at[pl.ds(0, CR)]], rows_v.at[b], gsem.at[b]
            ).wait()

        def s_start(b, j):
            pltpu.async_copy(
                rows_v.at[b], out_hbm.at[pl.ds(base + j * CR, CR)], ssem.at[b]
            )

        def s_wait(b):
            pltpu.make_async_copy(
                rows_v.at[b], out_hbm.at[pl.ds(base, CR)], ssem.at[b]
            ).wait()

        for b in range(NB):
            g_start(b, b)
        for j in range(n_chunks):
            b = j % NB
            g_wait(b)
            s_start(b, j)
            if j + NB < n_chunks:
                s_wait(b)
                g_start(b, j + NB)
        for j in range(max(0, n_chunks - NB), n_chunks):
            s_wait(j % NB)

    return gather


def kernel(input, table):
    B, S = input.shape
    V, D = table.shape
    N = B * S
    NW = 32
    idx = input.reshape(N).astype(jnp.int32)
    out = _make_sc_gather(V, D, N, NW, _NB, _CR)(idx, table)
    return out.reshape(B, S, D)
